# baseline (speedup 1.0000x reference)
import functools

import jax
import jax.numpy as jnp
from jax import lax
from jax.experimental import pallas as pl
from jax.experimental.pallas import tpu as pltpu
from jax.experimental.pallas import tpu_sc as plsc

NUM_CORES = 2
NUM_SUBCORES = 16
NUM_WORKERS = NUM_CORES * NUM_SUBCORES
CB = 2
IDX_SPLIT = 128


@functools.lru_cache(maxsize=None)
def _make_kernel(batch: int, hist: int, vocab: int, dim: int):
    bpw = batch // NUM_WORKERS
    n_chunks = bpw // CB
    assert bpw % CB == 0
    segs = []
    off = 0
    while off < hist:
        ln = min(IDX_SPLIT, hist - off)
        segs.append((off, ln))
        off += ln
    mesh = plsc.VectorSubcoreMesh(
        core_axis_name="c", subcore_axis_name="s")

    @functools.partial(
        pl.kernel,
        mesh=mesh,
        out_type=jax.ShapeDtypeStruct((batch, hist, dim), jnp.float32),
        scratch_types=[
            pltpu.VMEM((bpw, hist), jnp.int32),
            pltpu.VMEM((CB, hist, dim), jnp.float32),
            pltpu.SemaphoreType.DMA,
        ],
        compiler_params=pltpu.CompilerParams(use_tc_tiling_on_sc=False),
    )
    def emb_kernel(idx_hbm, table_hbm, out_hbm, idx_v, rows_v, gsem):
        wid = lax.axis_index("s") * NUM_CORES + lax.axis_index("c")
        base = wid * bpw
        pltpu.sync_copy(idx_hbm.at[pl.ds(base, bpw), :], idx_v)

        @pl.loop(0, n_chunks)
        def _(ci):
            bb = ci * CB
            copies = []
            for i in range(CB):
                for off, ln in segs:
                    copies.append(pltpu.async_copy(
                        table_hbm.at[idx_v.at[bb + i, pl.ds(off, ln)]],
                        rows_v.at[i, pl.ds(off, ln), :],
                        gsem))
            for c in copies:
                c.wait()
            pltpu.sync_copy(rows_v, out_hbm.at[pl.ds(base + bb, CB), :, :])

    return emb_kernel


def kernel(index, table):
    batch, hist = index.shape
    vocab, dim = table.shape
    return _make_kernel(batch, hist, vocab, dim)(index, table)


# speedup vs baseline: 1.4097x; 1.4097x over previous
import functools

import jax
import jax.numpy as jnp
from jax import lax
from jax.experimental import pallas as pl
from jax.experimental.pallas import tpu as pltpu
from jax.experimental.pallas import tpu_sc as plsc

NUM_CORES = 2
NUM_SUBCORES = 16
NUM_WORKERS = NUM_CORES * NUM_SUBCORES
LANES = 16


@functools.lru_cache(maxsize=None)
def _make_kernel(batch: int, hist: int, vocab: int, dim: int):
    rounds = dim // NUM_WORKERS
    bt = batch // 128
    blocks = batch // LANES
    mesh = plsc.VectorSubcoreMesh(
        core_axis_name="c", subcore_axis_name="s")

    @functools.partial(
        pl.kernel,
        mesh=mesh,
        out_type=jax.ShapeDtypeStruct((hist, dim // 8, bt, 8, 128),
                                      jnp.float32),
        scratch_types=[
            pltpu.VMEM((vocab,), jnp.float32),
            pltpu.VMEM((2, batch), jnp.int32),
            pltpu.VMEM((2, bt, 128), jnp.float32),
            pltpu.SemaphoreType.DMA,
            pltpu.SemaphoreType.DMA,
            pltpu.SemaphoreType.DMA,
            pltpu.SemaphoreType.DMA,
        ],
        compiler_params=pltpu.CompilerParams(use_tc_tiling_on_sc=False,
                                             needs_layout_passes=False),
    )
    def emb_kernel(idx_t, table_t, out_k, feat_v, idx_v, out_v,
                   isem0, isem1, osem0, osem1):
        wid = lax.axis_index("s") * NUM_CORES + lax.axis_index("c")
        isems = (isem0, isem1)
        osems = (osem0, osem1)

        for r in range(rounds):
            d = wid * rounds + r
            dt = d // 8
            ds = d % 8
            pltpu.sync_copy(table_t.at[d, :], feat_v)
            for b in range(2):
                pltpu.async_copy(idx_t.at[b, :], idx_v.at[b, :], isems[b])

            @pl.loop(0, hist, step=2)
            def _(hh):
                for b in range(2):
                    h = hh + b
                    pltpu.make_async_copy(
                        idx_t.at[h, :], idx_v.at[b, :], isems[b]).wait()

                    def _drain():
                        pltpu.make_async_copy(
                            out_v.at[b], out_k.at[0, 0, :, 0, :],
                            osems[b]).wait()

                    if r > 0:
                        _drain()
                    else:
                        pl.when(h >= 2)(_drain)

                    @pl.loop(0, bt)
                    def _(jj):
                        for l in range(8):
                            col = jj * 8 + l
                            idx = idx_v[b, pl.ds(col * LANES, LANES)]
                            out_v[b, jj, pl.ds(l * LANES, LANES)] = (
                                plsc.load_gather(feat_v, [idx]))

                    pltpu.async_copy(
                        out_v.at[b], out_k.at[h, dt, :, ds, :], osems[b])

                    @pl.when(h + 2 < hist)
                    def _():
                        pltpu.async_copy(
                            idx_t.at[h + 2, :], idx_v.at[b, :], isems[b])

            if r == rounds - 1:
                for b in range(2):
                    pltpu.make_async_copy(
                        out_v.at[b], out_k.at[0, 0, :, 0, :],
                        osems[b]).wait()

    return emb_kernel


def kernel(index, table):
    batch, hist = index.shape
    vocab, dim = table.shape
    out_k = _make_kernel(batch, hist, vocab, dim)(index.T, table.T)
    return out_k.transpose(2, 4, 0, 1, 3).reshape(batch, hist, dim)


# speedup vs baseline: 2.0240x; 1.4358x over previous
import functools

import jax
import jax.numpy as jnp
from jax import lax
from jax.experimental import pallas as pl
from jax.experimental.pallas import tpu as pltpu
from jax.experimental.pallas import tpu_sc as plsc

NUM_CORES = 2
NUM_SUBCORES = 16
NUM_WORKERS = NUM_CORES * NUM_SUBCORES
LANES = 16


@functools.lru_cache(maxsize=None)
def _make_kernel(batch: int, hist: int, vocab: int, dim: int):
    rounds = dim // NUM_WORKERS
    bt = batch // 128
    blocks = batch // LANES
    mesh = plsc.VectorSubcoreMesh(
        core_axis_name="c", subcore_axis_name="s")

    @functools.partial(
        pl.kernel,
        mesh=mesh,
        out_type=jax.ShapeDtypeStruct((hist, dim // 8, bt, 8, 128),
                                      jnp.float32),
        scratch_types=[
            pltpu.VMEM((vocab,), jnp.float32),
            pltpu.VMEM((2, batch), jnp.int32),
            pltpu.VMEM((2, bt, 128), jnp.float32),
            pltpu.SemaphoreType.DMA,
            pltpu.SemaphoreType.DMA,
            pltpu.SemaphoreType.DMA,
            pltpu.SemaphoreType.DMA,
        ],
        compiler_params=pltpu.CompilerParams(use_tc_tiling_on_sc=False,
                                             needs_layout_passes=False),
    )
    def emb_kernel(idx_t, table_t, out_k, feat_v, idx_v, out_v,
                   isem0, isem1, osem0, osem1):
        wid = lax.axis_index("s") * NUM_CORES + lax.axis_index("c")
        isems = (isem0, isem1)
        osems = (osem0, osem1)

        for r in range(rounds):
            d = wid * rounds + r
            dt = d // 8
            ds = d % 8
            pltpu.sync_copy(table_t.at[d, :], feat_v)
            for b in range(2):
                pltpu.async_copy(idx_t.at[b, :], idx_v.at[b, :], isems[b])

            @pl.loop(0, hist, step=2)
            def _(hh):
                for b in range(2):
                    h = hh + b
                    pltpu.make_async_copy(
                        idx_t.at[h, :], idx_v.at[b, :], isems[b]).wait()

                    def _drain():
                        pltpu.make_async_copy(
                            out_v.at[b], out_k.at[0, 0, :, 0, :],
                            osems[b]).wait()

                    if r > 0:
                        _drain()
                    else:
                        pl.when(h >= 2)(_drain)

                    @plsc.parallel_loop(0, blocks, unroll=8)
                    def _(j):
                        idx = idx_v[b, pl.ds(j * LANES, LANES)]
                        out_v[b, j // 8, pl.ds((j % 8) * LANES, LANES)] = (
                            plsc.load_gather(feat_v, [idx]))

                    pltpu.async_copy(
                        out_v.at[b], out_k.at[h, dt, :, ds, :], osems[b])

                    @pl.when(h + 2 < hist)
                    def _():
                        pltpu.async_copy(
                            idx_t.at[h + 2, :], idx_v.at[b, :], isems[b])

            if r == rounds - 1:
                for b in range(2):
                    pltpu.make_async_copy(
                        out_v.at[b], out_k.at[0, 0, :, 0, :],
                        osems[b]).wait()

    return emb_kernel


def kernel(index, table):
    batch, hist = index.shape
    vocab, dim = table.shape
    out_k = _make_kernel(batch, hist, vocab, dim)(index.T, table.T)
    return out_k.transpose(2, 4, 0, 1, 3).reshape(batch, hist, dim)
